# Initial kernel scaffold; baseline (speedup 1.0000x reference)
import jax
import jax.numpy as jnp
from jax.experimental import pallas as pl


def kernel(x, quotient_table, remainder_table):
    raise NotImplementedError("write your pallas kernel here")


# speedup vs baseline: 5.0096x; 5.0096x over previous
import functools

import jax
import jax.numpy as jnp
from jax import lax
from jax.experimental import pallas as pl
from jax.experimental.pallas import tpu as pltpu
from jax.experimental.pallas import tpu_sc as plsc

MOD = 100
NB, NS_SEQ = 16384, 26
N_TOTAL = NB * NS_SEQ
NC, NSC, LANES = 2, 16, 16
NW = NC * NSC
PER_W = N_TOTAL // NW
CHUNK = 512
N_CHUNKS = PER_W // CHUNK
D = 16

assert N_TOTAL % NW == 0 and PER_W % CHUNK == 0


def _sc_lookup_body(x_hbm, qt_hbm, rt_hbm, out_hbm,
                    x_v, qi_v, ri_v, qr_v, rr_v, o_v, gsem):
    wid = lax.axis_index("s") * NC + lax.axis_index("c")
    base = wid * PER_W

    pltpu.sync_copy(x_hbm.at[pl.ds(base, PER_W)], x_v)

    def divmod_body(i, carry):
        v = x_v[pl.ds(i * LANES, LANES)]
        q = lax.div(v, MOD)
        qi_v[pl.ds(i * LANES, LANES)] = q
        ri_v[pl.ds(i * LANES, LANES)] = v - q * MOD
        return carry

    lax.fori_loop(0, PER_W // LANES, divmod_body, 0)

    def chunk_body(j, carry):
        cq = pltpu.make_async_copy(
            qt_hbm.at[qi_v.at[pl.ds(j * CHUNK, CHUNK)]], qr_v, gsem)
        cr = pltpu.make_async_copy(
            rt_hbm.at[ri_v.at[pl.ds(j * CHUNK, CHUNK)]], rr_v, gsem)
        cq.start()
        cr.start()
        cq.wait()
        cr.wait()

        def mul_body(i, c):
            o_v[i, :] = qr_v[i, :] * rr_v[i, :]
            return c

        lax.fori_loop(0, CHUNK, mul_body, 0)
        pltpu.sync_copy(o_v, out_hbm.at[pl.ds(base + j * CHUNK, CHUNK)])
        return carry

    lax.fori_loop(0, N_CHUNKS, chunk_body, 0)


@functools.partial(
    pl.kernel,
    out_type=jax.ShapeDtypeStruct((N_TOTAL, D), jnp.float32),
    mesh=plsc.VectorSubcoreMesh(core_axis_name="c", subcore_axis_name="s"),
    compiler_params=pltpu.CompilerParams(use_tc_tiling_on_sc=False),
    scratch_types=[
        pltpu.VMEM((PER_W,), jnp.int32),
        pltpu.VMEM((PER_W,), jnp.int32),
        pltpu.VMEM((PER_W,), jnp.int32),
        pltpu.VMEM((CHUNK, D), jnp.float32),
        pltpu.VMEM((CHUNK, D), jnp.float32),
        pltpu.VMEM((CHUNK, D), jnp.float32),
        pltpu.SemaphoreType.DMA,
    ],
)
def _sc_lookup(*refs):
    _sc_lookup_body(*refs)


def kernel(x, quotient_table, remainder_table):
    x_flat = x.reshape(N_TOTAL).astype(jnp.int32)
    out = _sc_lookup(x_flat, quotient_table, remainder_table)
    return out.reshape(NB, NS_SEQ, D)


# speedup vs baseline: 7.1953x; 1.4363x over previous
import functools

import jax
import jax.numpy as jnp
from jax import lax
from jax.experimental import pallas as pl
from jax.experimental.pallas import tpu as pltpu
from jax.experimental.pallas import tpu_sc as plsc

MOD = 100
NB, NS_SEQ = 16384, 26
N_TOTAL = NB * NS_SEQ
NC, NSC, LANES = 2, 16, 16
NW = NC * NSC
PER_W = N_TOTAL // NW
GROWS_W = NB // NW
CHUNK_G = 16
CHUNK = CHUNK_G * NS_SEQ
N_CHUNKS = PER_W // CHUNK
D = 16

assert N_TOTAL % NW == 0 and PER_W % CHUNK == 0 and N_CHUNKS % 2 == 0


def _sc_lookup_body(x_hbm, qt_hbm, rt_hbm, out_hbm,
                    x_v, qi_v, ri_v, qr0, rr0, qr1, rr1, o0, o1,
                    g0, g1, s0, s1):
    wid = lax.axis_index("s") * NC + lax.axis_index("c")
    base = wid * PER_W
    gbase = wid * GROWS_W

    pltpu.sync_copy(x_hbm.at[pl.ds(base, PER_W)], x_v)

    def divmod_body(i, carry):
        v = x_v[pl.ds(i * LANES, LANES)]
        q = lax.div(v, MOD)
        qi_v[pl.ds(i * LANES, LANES)] = q
        ri_v[pl.ds(i * LANES, LANES)] = v - q * MOD
        return carry

    lax.fori_loop(0, PER_W // LANES, divmod_body, 0)

    def start_gather(j, qr, rr, gsem):
        pltpu.make_async_copy(
            qt_hbm.at[qi_v.at[pl.ds(j * CHUNK, CHUNK)]], qr, gsem).start()
        pltpu.make_async_copy(
            rt_hbm.at[ri_v.at[pl.ds(j * CHUNK, CHUNK)]], rr, gsem).start()

    def wait_gather(qr, rr, gsem):
        pltpu.make_async_copy(
            qt_hbm.at[qi_v.at[pl.ds(0, CHUNK)]], qr, gsem).wait()
        pltpu.make_async_copy(
            rt_hbm.at[ri_v.at[pl.ds(0, CHUNK)]], rr, gsem).wait()

    def out_copy(j, o, osem):
        return pltpu.make_async_copy(
            o, out_hbm.at[pl.ds(gbase + j * CHUNK_G, CHUNK_G)], osem)

    def mul(qr, rr, o):
        def outer(a, carry):
            for b in range(NS_SEQ):
                i = a * NS_SEQ + b
                o[a, b, :] = qr[i, :] * rr[i, :]
            return carry
        lax.fori_loop(0, CHUNK_G, outer, 0)

    start_gather(0, qr0, rr0, g0)

    def step(t, carry):
        j0 = 2 * t
        start_gather(j0 + 1, qr1, rr1, g1)
        wait_gather(qr0, rr0, g0)

        @pl.when(t > 0)
        def _():
            out_copy(0, o0, s0).wait()

        mul(qr0, rr0, o0)
        out_copy(j0, o0, s0).start()

        @pl.when(t < N_CHUNKS // 2 - 1)
        def _():
            start_gather(j0 + 2, qr0, rr0, g0)

        wait_gather(qr1, rr1, g1)

        @pl.when(t > 0)
        def _():
            out_copy(0, o1, s1).wait()

        mul(qr1, rr1, o1)
        out_copy(j0 + 1, o1, s1).start()
        return carry

    lax.fori_loop(0, N_CHUNKS // 2, step, 0)
    out_copy(0, o0, s0).wait()
    out_copy(0, o1, s1).wait()


@functools.partial(
    pl.kernel,
    out_type=jax.ShapeDtypeStruct((NB, NS_SEQ, D), jnp.float32),
    mesh=plsc.VectorSubcoreMesh(core_axis_name="c", subcore_axis_name="s"),
    compiler_params=pltpu.CompilerParams(use_tc_tiling_on_sc=False),
    scratch_types=[
        pltpu.VMEM((PER_W,), jnp.int32),
        pltpu.VMEM((PER_W,), jnp.int32),
        pltpu.VMEM((PER_W,), jnp.int32),
        pltpu.VMEM((CHUNK, D), jnp.float32),
        pltpu.VMEM((CHUNK, D), jnp.float32),
        pltpu.VMEM((CHUNK, D), jnp.float32),
        pltpu.VMEM((CHUNK, D), jnp.float32),
        pltpu.VMEM((CHUNK_G, NS_SEQ, D), jnp.float32),
        pltpu.VMEM((CHUNK_G, NS_SEQ, D), jnp.float32),
        pltpu.SemaphoreType.DMA,
        pltpu.SemaphoreType.DMA,
        pltpu.SemaphoreType.DMA,
        pltpu.SemaphoreType.DMA,
    ],
)
def _sc_lookup(*refs):
    _sc_lookup_body(*refs)


def kernel(x, quotient_table, remainder_table):
    x_flat = x.reshape(N_TOTAL).astype(jnp.int32)
    return _sc_lookup(x_flat, quotient_table, remainder_table)


# speedup vs baseline: 11.5671x; 1.6076x over previous
import functools

import jax
import jax.numpy as jnp
from jax import lax
from jax.experimental import pallas as pl
from jax.experimental.pallas import tpu as pltpu
from jax.experimental.pallas import tpu_sc as plsc

MOD = 100
NB, NS_SEQ = 16384, 26
N_TOTAL = NB * NS_SEQ
NC, NSC, LANES = 2, 16, 16
NW = NC * NSC
PER_W = N_TOTAL // NW
GROWS_W = NB // NW
CHUNK_G = 16
CHUNK = CHUNK_G * NS_SEQ
N_CHUNKS = PER_W // CHUNK
D = 16

assert N_TOTAL % NW == 0 and PER_W % CHUNK == 0 and N_CHUNKS % 2 == 0


def _sc_lookup_body(x_hbm, qt_hbm, rt_hbm, out_hbm,
                    x_v, qi_v, ri_v, qr0, rr0, qr1, rr1, o0, o1,
                    qt_sh, rt_sh,
                    g0, g1, s0, s1):
    wid = lax.axis_index("s") * NC + lax.axis_index("c")
    base = wid * PER_W
    gbase = wid * GROWS_W

    @pl.when(lax.axis_index("s") == 0)
    def _():
        pltpu.sync_copy(qt_hbm, qt_sh)
        pltpu.sync_copy(rt_hbm, rt_sh)

    pltpu.sync_copy(x_hbm.at[pl.ds(base, PER_W)], x_v)

    def divmod_body(i, carry):
        v = x_v[pl.ds(i * LANES, LANES)]
        xf = v.astype(jnp.float32)
        q = ((xf + 0.5) * 0.01).astype(jnp.int32)
        qi_v[pl.ds(i * LANES, LANES)] = q
        ri_v[pl.ds(i * LANES, LANES)] = v - q * MOD
        return carry

    lax.fori_loop(0, PER_W // LANES, divmod_body, 0)
    plsc.subcore_barrier()

    def start_gather(j, qr, rr, gsem):
        pltpu.make_async_copy(
            qt_sh.at[qi_v.at[pl.ds(j * CHUNK, CHUNK)]], qr, gsem).start()
        pltpu.make_async_copy(
            rt_sh.at[ri_v.at[pl.ds(j * CHUNK, CHUNK)]], rr, gsem).start()

    def wait_gather(qr, rr, gsem):
        pltpu.make_async_copy(
            qt_sh.at[qi_v.at[pl.ds(0, CHUNK)]], qr, gsem).wait()
        pltpu.make_async_copy(
            rt_sh.at[ri_v.at[pl.ds(0, CHUNK)]], rr, gsem).wait()

    def out_copy(j, o, osem):
        return pltpu.make_async_copy(
            o, out_hbm.at[pl.ds(gbase + j * CHUNK_G, CHUNK_G)], osem)

    def mul(qr, rr, o):
        def outer(a, carry):
            for b in range(NS_SEQ):
                i = a * NS_SEQ + b
                o[a, b, :] = qr[i, :] * rr[i, :]
            return carry
        lax.fori_loop(0, CHUNK_G, outer, 0)

    start_gather(0, qr0, rr0, g0)

    def step(t, carry):
        j0 = 2 * t
        start_gather(j0 + 1, qr1, rr1, g1)
        wait_gather(qr0, rr0, g0)

        @pl.when(t > 0)
        def _():
            out_copy(0, o0, s0).wait()

        mul(qr0, rr0, o0)
        out_copy(j0, o0, s0).start()

        @pl.when(t < N_CHUNKS // 2 - 1)
        def _():
            start_gather(j0 + 2, qr0, rr0, g0)

        wait_gather(qr1, rr1, g1)

        @pl.when(t > 0)
        def _():
            out_copy(0, o1, s1).wait()

        mul(qr1, rr1, o1)
        out_copy(j0 + 1, o1, s1).start()
        return carry

    lax.fori_loop(0, N_CHUNKS // 2, step, 0)
    out_copy(0, o0, s0).wait()
    out_copy(0, o1, s1).wait()


@functools.partial(
    pl.kernel,
    out_type=jax.ShapeDtypeStruct((NB, NS_SEQ, D), jnp.float32),
    mesh=plsc.VectorSubcoreMesh(core_axis_name="c", subcore_axis_name="s"),
    compiler_params=pltpu.CompilerParams(use_tc_tiling_on_sc=False),
    scratch_types=[
        pltpu.VMEM((PER_W,), jnp.int32),
        pltpu.VMEM((PER_W,), jnp.int32),
        pltpu.VMEM((PER_W,), jnp.int32),
        pltpu.VMEM((CHUNK, D), jnp.float32),
        pltpu.VMEM((CHUNK, D), jnp.float32),
        pltpu.VMEM((CHUNK, D), jnp.float32),
        pltpu.VMEM((CHUNK, D), jnp.float32),
        pltpu.VMEM((CHUNK_G, NS_SEQ, D), jnp.float32),
        pltpu.VMEM((CHUNK_G, NS_SEQ, D), jnp.float32),
        pltpu.VMEM_SHARED((10001, D), jnp.float32),
        pltpu.VMEM_SHARED((MOD, D), jnp.float32),
        pltpu.SemaphoreType.DMA,
        pltpu.SemaphoreType.DMA,
        pltpu.SemaphoreType.DMA,
        pltpu.SemaphoreType.DMA,
    ],
)
def _sc_lookup(*refs):
    _sc_lookup_body(*refs)


def kernel(x, quotient_table, remainder_table):
    x_flat = x.reshape(N_TOTAL).astype(jnp.int32)
    return _sc_lookup(x_flat, quotient_table, remainder_table)
